# baseline (speedup 1.0000x reference)
import functools

import jax
import jax.numpy as jnp
from jax import lax
from jax.experimental import pallas as pl
from jax.experimental.pallas import tpu as pltpu
from jax.experimental.pallas import tpu_sc as plsc

BATCH = 16384
DIM = 128
NC = 2
NS = 16
NW = NC * NS
B_PER_W = BATCH // NW
CHUNK = 128
N_CHUNKS = B_PER_W // CHUNK
NBUF = 2
GRP = 8


def _body(heads_hbm, rels_hbm, tails_hbm, ent_hbm, rel_hbm, out_hbm,
          hidx, ridx, tidx, rows, outb, isem, sem0, sem1):
    wid = lax.axis_index("s") * NC + lax.axis_index("c")
    base = wid * B_PER_W
    lane = lax.iota(jnp.int32, 16)
    sems = (sem0, sem1)

    ci = pltpu.async_copy(heads_hbm.at[pl.ds(base, B_PER_W)], hidx, isem)
    cj = pltpu.async_copy(rels_hbm.at[pl.ds(base, B_PER_W)], ridx, isem)
    ck = pltpu.async_copy(tails_hbm.at[pl.ds(base, B_PER_W)], tidx, isem)
    ci.wait()
    cj.wait()
    ck.wait()

    def fire(c):
        buf = c % NBUF
        s = pl.ds(c * CHUNK, CHUNK)
        return (
            pltpu.async_copy(ent_hbm.at[hidx.at[s]], rows.at[buf, 0], sems[buf]),
            pltpu.async_copy(rel_hbm.at[ridx.at[s]], rows.at[buf, 1], sems[buf]),
            pltpu.async_copy(ent_hbm.at[tidx.at[s]], rows.at[buf, 2], sems[buf]),
        )

    inflight = fire(0)
    for c in range(N_CHUNKS):
        cur = inflight
        if c + 1 < N_CHUNKS:
            inflight = fire(c + 1)
        for d in cur:
            d.wait()
        buf = c % NBUF
        hrow = rows.at[buf, 0]
        rrow = rows.at[buf, 1]
        trow = rows.at[buf, 2]

        def group(g, carry):
            e0 = g * GRP
            res = jnp.zeros((16,), jnp.float32)
            for u in range(GRP):
                e = e0 + u
                acc_a = None
                acc_b = None
                for k in range(DIM // 16):
                    sl = pl.ds(k * 16, 16)
                    term = jnp.abs(hrow[e, sl] + rrow[e, sl] - trow[e, sl])
                    if k % 2 == 0:
                        acc_a = term if acc_a is None else acc_a + term
                    else:
                        acc_b = term if acc_b is None else acc_b + term
                tot = jnp.sum(acc_a + acc_b)
                res = jnp.where(lane == u, tot, res)
            plsc.store_scatter(outb, [c * CHUNK + e0 + lane], res,
                               mask=lane < GRP)
            return carry

        lax.fori_loop(0, CHUNK // GRP, group, 0)

    pltpu.sync_copy(outb, out_hbm.at[pl.ds(base, B_PER_W)])


@jax.jit
def kernel(heads, rels, tails, ent_embs, rel_embs):
    mesh = plsc.VectorSubcoreMesh(core_axis_name="c", subcore_axis_name="s")
    f = functools.partial(
        pl.kernel,
        mesh=mesh,
        compiler_params=pltpu.CompilerParams(needs_layout_passes=False),
        out_type=jax.ShapeDtypeStruct((BATCH,), jnp.float32),
        scratch_types=[
            pltpu.VMEM((B_PER_W,), jnp.int32),
            pltpu.VMEM((B_PER_W,), jnp.int32),
            pltpu.VMEM((B_PER_W,), jnp.int32),
            pltpu.VMEM((NBUF, 3, CHUNK, DIM), jnp.float32),
            pltpu.VMEM((B_PER_W,), jnp.float32),
            pltpu.SemaphoreType.DMA,
            pltpu.SemaphoreType.DMA,
            pltpu.SemaphoreType.DMA,
        ],
    )(_body)
    return f(heads, rels, tails, ent_embs, rel_embs)


# speedup vs baseline: 1.1601x; 1.1601x over previous
import functools

import jax
import jax.numpy as jnp
from jax import lax
from jax.experimental import pallas as pl
from jax.experimental.pallas import tpu as pltpu
from jax.experimental.pallas import tpu_sc as plsc

BATCH = 16384
DIM = 128
NC = 2
NS = 16
NW = NC * NS
B_PER_W = BATCH // NW
CHUNK = 128
N_CHUNKS = B_PER_W // CHUNK
NBUF = 2
GRP = 4


def _body(heads_hbm, rels_hbm, tails_hbm, ent_hbm, rel_hbm, out_hbm,
          hidx, ridx, tidx, rows, outb, isem, sem0, sem1):
    wid = lax.axis_index("s") * NC + lax.axis_index("c")
    base = wid * B_PER_W
    lane = lax.iota(jnp.int32, 16)
    sems = (sem0, sem1)

    ci = pltpu.async_copy(heads_hbm.at[pl.ds(base, B_PER_W)], hidx, isem)
    cj = pltpu.async_copy(rels_hbm.at[pl.ds(base, B_PER_W)], ridx, isem)
    ck = pltpu.async_copy(tails_hbm.at[pl.ds(base, B_PER_W)], tidx, isem)
    ci.wait()
    cj.wait()
    ck.wait()

    def fire(c):
        buf = c % NBUF
        s = pl.ds(c * CHUNK, CHUNK)
        return (
            pltpu.async_copy(ent_hbm.at[hidx.at[s]], rows.at[buf, 0], sems[buf]),
            pltpu.async_copy(rel_hbm.at[ridx.at[s]], rows.at[buf, 1], sems[buf]),
            pltpu.async_copy(ent_hbm.at[tidx.at[s]], rows.at[buf, 2], sems[buf]),
        )

    inflight = fire(0)
    for c in range(N_CHUNKS):
        cur = inflight
        if c + 1 < N_CHUNKS:
            inflight = fire(c + 1)
        for d in cur:
            d.wait()
        buf = c % NBUF
        hrow = rows.at[buf, 0]
        rrow = rows.at[buf, 1]
        trow = rows.at[buf, 2]

        def group(g, carry):
            e0 = g * GRP
            res = jnp.zeros((16,), jnp.float32)
            for u in range(GRP):
                e = e0 + u
                acc_a = None
                acc_b = None
                for k in range(DIM // 16):
                    sl = pl.ds(k * 16, 16)
                    term = jnp.abs(hrow[e, sl] + rrow[e, sl] - trow[e, sl])
                    if k % 2 == 0:
                        acc_a = term if acc_a is None else acc_a + term
                    else:
                        acc_b = term if acc_b is None else acc_b + term
                tot = jnp.sum(acc_a + acc_b)
                res = jnp.where(lane == u, tot, res)
            plsc.store_scatter(outb, [c * CHUNK + e0 + lane], res,
                               mask=lane < GRP)
            return carry

        lax.fori_loop(0, CHUNK // GRP, group, 0)

    pltpu.sync_copy(outb, out_hbm.at[pl.ds(base, B_PER_W)])


@jax.jit
def kernel(heads, rels, tails, ent_embs, rel_embs):
    mesh = plsc.VectorSubcoreMesh(core_axis_name="c", subcore_axis_name="s")
    f = functools.partial(
        pl.kernel,
        mesh=mesh,
        compiler_params=pltpu.CompilerParams(needs_layout_passes=False),
        out_type=jax.ShapeDtypeStruct((BATCH,), jnp.float32),
        scratch_types=[
            pltpu.VMEM((B_PER_W,), jnp.int32),
            pltpu.VMEM((B_PER_W,), jnp.int32),
            pltpu.VMEM((B_PER_W,), jnp.int32),
            pltpu.VMEM((NBUF, 3, CHUNK, DIM), jnp.float32),
            pltpu.VMEM((B_PER_W,), jnp.float32),
            pltpu.SemaphoreType.DMA,
            pltpu.SemaphoreType.DMA,
            pltpu.SemaphoreType.DMA,
        ],
    )(_body)
    return f(heads, rels, tails, ent_embs, rel_embs)


# speedup vs baseline: 1.1668x; 1.0058x over previous
import functools

import jax
import jax.numpy as jnp
from jax import lax
from jax.experimental import pallas as pl
from jax.experimental.pallas import tpu as pltpu
from jax.experimental.pallas import tpu_sc as plsc

BATCH = 16384
DIM = 128
NC = 2
NS = 16
NW = NC * NS
B_PER_W = BATCH // NW
CHUNK = 128
N_CHUNKS = B_PER_W // CHUNK
NBUF = 2
GRP = 4


def _body(heads_hbm, rels_hbm, tails_hbm, ent_hbm, rel_hbm, out_hbm,
          hidx, ridx, tidx, rows, outb, isem, sem0, sem1):
    wid = lax.axis_index("s") * NC + lax.axis_index("c")
    base = wid * B_PER_W
    lane = lax.iota(jnp.int32, 16)
    sems = (sem0, sem1)

    ci = pltpu.async_copy(heads_hbm.at[pl.ds(base, B_PER_W)], hidx, isem)
    cj = pltpu.async_copy(rels_hbm.at[pl.ds(base, B_PER_W)], ridx, isem)
    ck = pltpu.async_copy(tails_hbm.at[pl.ds(base, B_PER_W)], tidx, isem)
    ci.wait()
    cj.wait()
    ck.wait()

    def fire(c):
        buf = c % NBUF
        s = pl.ds(c * CHUNK, CHUNK)
        return (
            pltpu.async_copy(ent_hbm.at[hidx.at[s]], rows.at[buf, 0], sems[buf]),
            pltpu.async_copy(rel_hbm.at[ridx.at[s]], rows.at[buf, 1], sems[buf]),
            pltpu.async_copy(ent_hbm.at[tidx.at[s]], rows.at[buf, 2], sems[buf]),
        )

    inflight = fire(0)
    for c in range(N_CHUNKS):
        cur = inflight
        if c + 1 < N_CHUNKS:
            inflight = fire(c + 1)
        for d in cur:
            d.wait()
        buf = c % NBUF
        hrow = rows.at[buf, 0]
        rrow = rows.at[buf, 1]
        trow = rows.at[buf, 2]

        @plsc.parallel_loop(0, CHUNK // GRP, unroll=2)
        def group(g):
            e0 = g * GRP
            res = jnp.zeros((16,), jnp.float32)
            for u in range(GRP):
                e = e0 + u
                acc_a = None
                acc_b = None
                for k in range(DIM // 16):
                    sl = pl.ds(k * 16, 16)
                    term = jnp.abs(hrow[e, sl] + rrow[e, sl] - trow[e, sl])
                    if k % 2 == 0:
                        acc_a = term if acc_a is None else acc_a + term
                    else:
                        acc_b = term if acc_b is None else acc_b + term
                tot = jnp.sum(acc_a + acc_b)
                res = jnp.where(lane == u, tot, res)
            plsc.store_scatter(outb, [c * CHUNK + e0 + lane], res,
                               mask=lane < GRP)

    pltpu.sync_copy(outb, out_hbm.at[pl.ds(base, B_PER_W)])


@jax.jit
def kernel(heads, rels, tails, ent_embs, rel_embs):
    mesh = plsc.VectorSubcoreMesh(core_axis_name="c", subcore_axis_name="s")
    f = functools.partial(
        pl.kernel,
        mesh=mesh,
        compiler_params=pltpu.CompilerParams(needs_layout_passes=False),
        out_type=jax.ShapeDtypeStruct((BATCH,), jnp.float32),
        scratch_types=[
            pltpu.VMEM((B_PER_W,), jnp.int32),
            pltpu.VMEM((B_PER_W,), jnp.int32),
            pltpu.VMEM((B_PER_W,), jnp.int32),
            pltpu.VMEM((NBUF, 3, CHUNK, DIM), jnp.float32),
            pltpu.VMEM((B_PER_W,), jnp.float32),
            pltpu.SemaphoreType.DMA,
            pltpu.SemaphoreType.DMA,
            pltpu.SemaphoreType.DMA,
        ],
    )(_body)
    return f(heads, rels, tails, ent_embs, rel_embs)


# speedup vs baseline: 1.1844x; 1.0151x over previous
import functools

import jax
import jax.numpy as jnp
from jax import lax
from jax.experimental import pallas as pl
from jax.experimental.pallas import tpu as pltpu
from jax.experimental.pallas import tpu_sc as plsc

BATCH = 16384
DIM = 128
NC = 2
NS = 16
NW = NC * NS
B_PER_W = BATCH // NW
CHUNK = 128
N_CHUNKS = B_PER_W // CHUNK
NBUF = 2
GRP = 4


def _body(heads_hbm, rels_hbm, tails_hbm, ent_hbm, rel_hbm, out_hbm,
          hidx, ridx, tidx, rows, outb, isem, sem0, sem1):
    wid = lax.axis_index("s") * NC + lax.axis_index("c")
    base = wid * B_PER_W
    lane = lax.iota(jnp.int32, 16)
    sems = (sem0, sem1)

    ci = pltpu.async_copy(heads_hbm.at[pl.ds(base, B_PER_W)], hidx, isem)
    cj = pltpu.async_copy(rels_hbm.at[pl.ds(base, B_PER_W)], ridx, isem)
    ck = pltpu.async_copy(tails_hbm.at[pl.ds(base, B_PER_W)], tidx, isem)
    ci.wait()
    cj.wait()
    ck.wait()

    def fire(c):
        buf = c % NBUF
        s = pl.ds(c * CHUNK, CHUNK)
        return (
            pltpu.async_copy(ent_hbm.at[hidx.at[s]], rows.at[buf, 0], sems[buf]),
            pltpu.async_copy(rel_hbm.at[ridx.at[s]], rows.at[buf, 1], sems[buf]),
            pltpu.async_copy(ent_hbm.at[tidx.at[s]], rows.at[buf, 2], sems[buf]),
        )

    inflight = fire(0)
    for c in range(N_CHUNKS):
        for d in inflight:
            d.wait()
        if c + 1 < N_CHUNKS:
            inflight = fire(c + 1)
        buf = c % NBUF
        hrow = rows.at[buf, 0]
        rrow = rows.at[buf, 1]
        trow = rows.at[buf, 2]

        @plsc.parallel_loop(0, CHUNK // GRP, unroll=2)
        def group(g):
            e0 = g * GRP
            res = jnp.zeros((16,), jnp.float32)
            for u in range(GRP):
                e = e0 + u
                acc_a = None
                acc_b = None
                for k in range(DIM // 16):
                    sl = pl.ds(k * 16, 16)
                    term = jnp.abs(hrow[e, sl] + rrow[e, sl] - trow[e, sl])
                    if k % 2 == 0:
                        acc_a = term if acc_a is None else acc_a + term
                    else:
                        acc_b = term if acc_b is None else acc_b + term
                tot = jnp.sum(acc_a + acc_b)
                res = jnp.where(lane == u, tot, res)
            plsc.store_scatter(outb, [c * CHUNK + e0 + lane], res,
                               mask=lane < GRP)

    pltpu.sync_copy(outb, out_hbm.at[pl.ds(base, B_PER_W)])


@jax.jit
def kernel(heads, rels, tails, ent_embs, rel_embs):
    mesh = plsc.VectorSubcoreMesh(core_axis_name="c", subcore_axis_name="s")
    f = functools.partial(
        pl.kernel,
        mesh=mesh,
        compiler_params=pltpu.CompilerParams(needs_layout_passes=False),
        out_type=jax.ShapeDtypeStruct((BATCH,), jnp.float32),
        scratch_types=[
            pltpu.VMEM((B_PER_W,), jnp.int32),
            pltpu.VMEM((B_PER_W,), jnp.int32),
            pltpu.VMEM((B_PER_W,), jnp.int32),
            pltpu.VMEM((NBUF, 3, CHUNK, DIM), jnp.float32),
            pltpu.VMEM((B_PER_W,), jnp.float32),
            pltpu.SemaphoreType.DMA,
            pltpu.SemaphoreType.DMA,
            pltpu.SemaphoreType.DMA,
        ],
    )(_body)
    return f(heads, rels, tails, ent_embs, rel_embs)


# speedup vs baseline: 1.1875x; 1.0026x over previous
import functools

import jax
import jax.numpy as jnp
from jax import lax
from jax.experimental import pallas as pl
from jax.experimental.pallas import tpu as pltpu
from jax.experimental.pallas import tpu_sc as plsc

BATCH = 16384
DIM = 128
NC = 2
NS = 16
NW = NC * NS
B_PER_W = BATCH // NW
CHUNK = 128
N_CHUNKS = B_PER_W // CHUNK
NBUF = 2
GRP = 4


def _body(heads_hbm, rels_hbm, tails_hbm, ent_hbm, rel_hbm, out_hbm,
          hidx, ridx, tidx, rows, outb, isem, sem0, sem1):
    wid = lax.axis_index("s") * NC + lax.axis_index("c")
    base = wid * B_PER_W
    lane = lax.iota(jnp.int32, 16)
    sems = (sem0, sem1)

    s0 = pl.ds(base, CHUNK)
    rest = pl.ds(base + CHUNK, B_PER_W - CHUNK)
    ci = pltpu.async_copy(heads_hbm.at[s0], hidx.at[pl.ds(0, CHUNK)], isem)
    cj = pltpu.async_copy(rels_hbm.at[s0], ridx.at[pl.ds(0, CHUNK)], isem)
    ck = pltpu.async_copy(tails_hbm.at[s0], tidx.at[pl.ds(0, CHUNK)], isem)
    ci.wait()
    cj.wait()
    ck.wait()

    def fire(c):
        buf = c % NBUF
        s = pl.ds(c * CHUNK, CHUNK)
        return (
            pltpu.async_copy(ent_hbm.at[hidx.at[s]], rows.at[buf, 0], sems[buf]),
            pltpu.async_copy(rel_hbm.at[ridx.at[s]], rows.at[buf, 1], sems[buf]),
            pltpu.async_copy(ent_hbm.at[tidx.at[s]], rows.at[buf, 2], sems[buf]),
        )

    inflight = fire(0)
    ca = pltpu.async_copy(heads_hbm.at[rest], hidx.at[pl.ds(CHUNK, B_PER_W - CHUNK)], isem)
    cb = pltpu.async_copy(rels_hbm.at[rest], ridx.at[pl.ds(CHUNK, B_PER_W - CHUNK)], isem)
    cc = pltpu.async_copy(tails_hbm.at[rest], tidx.at[pl.ds(CHUNK, B_PER_W - CHUNK)], isem)
    ca.wait()
    cb.wait()
    cc.wait()
    outcps = []
    for c in range(N_CHUNKS):
        for d in inflight:
            d.wait()
        if c + 1 < N_CHUNKS:
            inflight = fire(c + 1)
        buf = c % NBUF
        hrow = rows.at[buf, 0]
        rrow = rows.at[buf, 1]
        trow = rows.at[buf, 2]

        @plsc.parallel_loop(0, CHUNK // GRP, unroll=2)
        def group(g):
            e0 = g * GRP
            res = jnp.zeros((16,), jnp.float32)
            for u in range(GRP):
                e = e0 + u
                acc_a = None
                acc_b = None
                for k in range(DIM // 16):
                    sl = pl.ds(k * 16, 16)
                    term = jnp.abs(hrow[e, sl] + rrow[e, sl] - trow[e, sl])
                    if k % 2 == 0:
                        acc_a = term if acc_a is None else acc_a + term
                    else:
                        acc_b = term if acc_b is None else acc_b + term
                tot = jnp.sum(acc_a + acc_b)
                res = jnp.where(lane == u, tot, res)
            plsc.store_scatter(outb, [c * CHUNK + e0 + lane], res,
                               mask=lane < GRP)

        outcps.append(pltpu.async_copy(
            outb.at[pl.ds(c * CHUNK, CHUNK)],
            out_hbm.at[pl.ds(base + c * CHUNK, CHUNK)], isem))

    for d in outcps:
        d.wait()


@jax.jit
def kernel(heads, rels, tails, ent_embs, rel_embs):
    mesh = plsc.VectorSubcoreMesh(core_axis_name="c", subcore_axis_name="s")
    f = functools.partial(
        pl.kernel,
        mesh=mesh,
        compiler_params=pltpu.CompilerParams(needs_layout_passes=False),
        out_type=jax.ShapeDtypeStruct((BATCH,), jnp.float32),
        scratch_types=[
            pltpu.VMEM((B_PER_W,), jnp.int32),
            pltpu.VMEM((B_PER_W,), jnp.int32),
            pltpu.VMEM((B_PER_W,), jnp.int32),
            pltpu.VMEM((NBUF, 3, CHUNK, DIM), jnp.float32),
            pltpu.VMEM((B_PER_W,), jnp.float32),
            pltpu.SemaphoreType.DMA,
            pltpu.SemaphoreType.DMA,
            pltpu.SemaphoreType.DMA,
        ],
    )(_body)
    return f(heads, rels, tails, ent_embs, rel_embs)


# speedup vs baseline: 1.2563x; 1.0580x over previous
import functools

import jax
import jax.numpy as jnp
from jax import lax
from jax.experimental import pallas as pl
from jax.experimental.pallas import tpu as pltpu
from jax.experimental.pallas import tpu_sc as plsc

BATCH = 16384
DIM = 128
NC = 2
NS = 16
NW = NC * NS
B_PER_W = BATCH // NW
CHUNK = 128
N_CHUNKS = B_PER_W // CHUNK
NBUF = 3
GRP = 4


def _body(heads_hbm, rels_hbm, tails_hbm, ent_hbm, rel_hbm, out_hbm,
          hidx, ridx, tidx, rows, outb, isem,
          hs0, hs1, hs2, ts0, ts1, ts2, rs0, rs1, rs2):
    wid = lax.axis_index("s") * NC + lax.axis_index("c")
    base = wid * B_PER_W
    lane = lax.iota(jnp.int32, 16)
    hsems = (hs0, hs1, hs2)
    tsems = (ts0, ts1, ts2)
    rsems = (rs0, rs1, rs2)

    ci = pltpu.async_copy(heads_hbm.at[pl.ds(base, B_PER_W)], hidx, isem)
    cj = pltpu.async_copy(rels_hbm.at[pl.ds(base, B_PER_W)], ridx, isem)
    ck = pltpu.async_copy(tails_hbm.at[pl.ds(base, B_PER_W)], tidx, isem)
    ci.wait()
    cj.wait()
    ck.wait()

    def fire_ht(c):
        buf = c % NBUF
        s = pl.ds(c * CHUNK, CHUNK)
        return (
            pltpu.async_copy(ent_hbm.at[hidx.at[s]], rows.at[buf, 0], hsems[buf]),
            pltpu.async_copy(ent_hbm.at[tidx.at[s]], rows.at[buf, 1], tsems[buf]),
        )

    def fire_radd(c):
        buf = c % NBUF
        s = pl.ds(c * CHUNK, CHUNK)
        return pltpu.async_copy(rel_hbm.at[ridx.at[s]], rows.at[buf, 0],
                                rsems[buf], add=True)

    ht = {0: fire_ht(0), 1: fire_ht(1)}
    ht[0][0].wait()
    radd = {0: fire_radd(0)}

    outcps = []
    for c in range(N_CHUNKS):
        radd[c].wait()
        ht[c][1].wait()
        if c + 2 < N_CHUNKS:
            ht[c + 2] = fire_ht(c + 2)
        if c + 1 < N_CHUNKS:
            ht[c + 1][0].wait()
            radd[c + 1] = fire_radd(c + 1)
        buf = c % NBUF
        hrow = rows.at[buf, 0]
        trow = rows.at[buf, 1]

        @plsc.parallel_loop(0, CHUNK // GRP, unroll=2)
        def group(g):
            e0 = g * GRP
            res = jnp.zeros((16,), jnp.float32)
            for u in range(GRP):
                e = e0 + u
                acc_a = None
                acc_b = None
                for k in range(DIM // 16):
                    sl = pl.ds(k * 16, 16)
                    term = jnp.abs(hrow[e, sl] - trow[e, sl])
                    if k % 2 == 0:
                        acc_a = term if acc_a is None else acc_a + term
                    else:
                        acc_b = term if acc_b is None else acc_b + term
                tot = jnp.sum(acc_a + acc_b)
                res = jnp.where(lane == u, tot, res)
            plsc.store_scatter(outb, [c * CHUNK + e0 + lane], res,
                               mask=lane < GRP)

        outcps.append(pltpu.async_copy(
            outb.at[pl.ds(c * CHUNK, CHUNK)],
            out_hbm.at[pl.ds(base + c * CHUNK, CHUNK)], isem))

    for d in outcps:
        d.wait()


@jax.jit
def kernel(heads, rels, tails, ent_embs, rel_embs):
    mesh = plsc.VectorSubcoreMesh(core_axis_name="c", subcore_axis_name="s")
    f = functools.partial(
        pl.kernel,
        mesh=mesh,
        compiler_params=pltpu.CompilerParams(needs_layout_passes=False),
        out_type=jax.ShapeDtypeStruct((BATCH,), jnp.float32),
        scratch_types=[
            pltpu.VMEM((B_PER_W,), jnp.int32),
            pltpu.VMEM((B_PER_W,), jnp.int32),
            pltpu.VMEM((B_PER_W,), jnp.int32),
            pltpu.VMEM((NBUF, 2, CHUNK, DIM), jnp.float32),
            pltpu.VMEM((B_PER_W,), jnp.float32),
            pltpu.SemaphoreType.DMA,
            pltpu.SemaphoreType.DMA,
            pltpu.SemaphoreType.DMA,
            pltpu.SemaphoreType.DMA,
            pltpu.SemaphoreType.DMA,
            pltpu.SemaphoreType.DMA,
            pltpu.SemaphoreType.DMA,
            pltpu.SemaphoreType.DMA,
            pltpu.SemaphoreType.DMA,
            pltpu.SemaphoreType.DMA,
        ],
    )(_body)
    return f(heads, rels, tails, ent_embs, rel_embs)


# speedup vs baseline: 1.3003x; 1.0350x over previous
import functools

import jax
import jax.numpy as jnp
from jax import lax
from jax.experimental import pallas as pl
from jax.experimental.pallas import tpu as pltpu
from jax.experimental.pallas import tpu_sc as plsc

BATCH = 16384
DIM = 128
NC = 2
NS = 16
NW = NC * NS
B_PER_W = BATCH // NW
CHUNK = 128
N_CHUNKS = B_PER_W // CHUNK
NBUF = 3
GRP = 4


def _body(heads_hbm, rels_hbm, tails_hbm, ent_hbm, rel_hbm, out_hbm,
          hidx, ridx, tidx, rows, rtab, outb, isem,
          hs0, hs1, hs2, ts0, ts1, ts2, rs0, rs1, rs2):
    wid = lax.axis_index("s") * NC + lax.axis_index("c")
    base = wid * B_PER_W
    lane = lax.iota(jnp.int32, 16)
    hsems = (hs0, hs1, hs2)
    tsems = (ts0, ts1, ts2)
    rsems = (rs0, rs1, rs2)

    @pl.when(lax.axis_index("s") == 0)
    def _stage_rel():
        pltpu.sync_copy(rel_hbm, rtab)

    ci = pltpu.async_copy(heads_hbm.at[pl.ds(base, B_PER_W)], hidx, isem)
    cj = pltpu.async_copy(rels_hbm.at[pl.ds(base, B_PER_W)], ridx, isem)
    ck = pltpu.async_copy(tails_hbm.at[pl.ds(base, B_PER_W)], tidx, isem)
    ci.wait()
    cj.wait()
    ck.wait()
    plsc.subcore_barrier()

    def fire_ht(c):
        buf = c % NBUF
        s = pl.ds(c * CHUNK, CHUNK)
        return (
            pltpu.async_copy(ent_hbm.at[hidx.at[s]], rows.at[buf, 0], hsems[buf]),
            pltpu.async_copy(ent_hbm.at[tidx.at[s]], rows.at[buf, 1], tsems[buf]),
        )

    def fire_radd(c):
        buf = c % NBUF
        s = pl.ds(c * CHUNK, CHUNK)
        return pltpu.async_copy(rtab.at[ridx.at[s]], rows.at[buf, 0],
                                rsems[buf], add=True)

    ht = {0: fire_ht(0), 1: fire_ht(1)}
    ht[0][0].wait()
    radd = {0: fire_radd(0)}

    outcps = []
    for c in range(N_CHUNKS):
        radd[c].wait()
        ht[c][1].wait()
        if c + 2 < N_CHUNKS:
            ht[c + 2] = fire_ht(c + 2)
        if c + 1 < N_CHUNKS:
            ht[c + 1][0].wait()
            radd[c + 1] = fire_radd(c + 1)
        buf = c % NBUF
        hrow = rows.at[buf, 0]
        trow = rows.at[buf, 1]

        @plsc.parallel_loop(0, CHUNK // GRP, unroll=2)
        def group(g):
            e0 = g * GRP
            res = jnp.zeros((16,), jnp.float32)
            for u in range(GRP):
                e = e0 + u
                acc_a = None
                acc_b = None
                for k in range(DIM // 16):
                    sl = pl.ds(k * 16, 16)
                    term = jnp.abs(hrow[e, sl] - trow[e, sl])
                    if k % 2 == 0:
                        acc_a = term if acc_a is None else acc_a + term
                    else:
                        acc_b = term if acc_b is None else acc_b + term
                tot = jnp.sum(acc_a + acc_b)
                res = jnp.where(lane == u, tot, res)
            plsc.store_scatter(outb, [c * CHUNK + e0 + lane], res,
                               mask=lane < GRP)

        outcps.append(pltpu.async_copy(
            outb.at[pl.ds(c * CHUNK, CHUNK)],
            out_hbm.at[pl.ds(base + c * CHUNK, CHUNK)], isem))

    for d in outcps:
        d.wait()


@jax.jit
def kernel(heads, rels, tails, ent_embs, rel_embs):
    mesh = plsc.VectorSubcoreMesh(core_axis_name="c", subcore_axis_name="s")
    f = functools.partial(
        pl.kernel,
        mesh=mesh,
        compiler_params=pltpu.CompilerParams(needs_layout_passes=False),
        out_type=jax.ShapeDtypeStruct((BATCH,), jnp.float32),
        scratch_types=[
            pltpu.VMEM((B_PER_W,), jnp.int32),
            pltpu.VMEM((B_PER_W,), jnp.int32),
            pltpu.VMEM((B_PER_W,), jnp.int32),
            pltpu.VMEM((NBUF, 2, CHUNK, DIM), jnp.float32),
            pltpu.VMEM_SHARED((1000, DIM), jnp.float32),
            pltpu.VMEM((B_PER_W,), jnp.float32),
            pltpu.SemaphoreType.DMA,
            pltpu.SemaphoreType.DMA,
            pltpu.SemaphoreType.DMA,
            pltpu.SemaphoreType.DMA,
            pltpu.SemaphoreType.DMA,
            pltpu.SemaphoreType.DMA,
            pltpu.SemaphoreType.DMA,
            pltpu.SemaphoreType.DMA,
            pltpu.SemaphoreType.DMA,
            pltpu.SemaphoreType.DMA,
        ],
    )(_body)
    return f(heads, rels, tails, ent_embs, rel_embs)


# speedup vs baseline: 1.3110x; 1.0082x over previous
import functools

import jax
import jax.numpy as jnp
from jax import lax
from jax.experimental import pallas as pl
from jax.experimental.pallas import tpu as pltpu
from jax.experimental.pallas import tpu_sc as plsc

BATCH = 16384
DIM = 128
NC = 2
NS = 16
NW = NC * NS
B_PER_W = BATCH // NW
CHUNK = 128
CHUNKS = ((0, 64), (64, 64), (128, 128), (256, 128), (384, 128))
NBUF = 3
GRP = 4


def _body(heads_hbm, rels_hbm, tails_hbm, ent_hbm, rel_hbm, out_hbm,
          hidx, ridx, tidx, rows, rtab, outb, isem,
          hs0, hs1, hs2, ts0, ts1, ts2, rs0, rs1, rs2):
    wid = lax.axis_index("s") * NC + lax.axis_index("c")
    base = wid * B_PER_W
    lane = lax.iota(jnp.int32, 16)
    hsems = (hs0, hs1, hs2)
    tsems = (ts0, ts1, ts2)
    rsems = (rs0, rs1, rs2)

    ci = pltpu.async_copy(heads_hbm.at[pl.ds(base, B_PER_W)], hidx, isem)
    cj = pltpu.async_copy(rels_hbm.at[pl.ds(base, B_PER_W)], ridx, isem)
    ck = pltpu.async_copy(tails_hbm.at[pl.ds(base, B_PER_W)], tidx, isem)

    rtab_cp = []

    @pl.when(lax.axis_index("s") == 0)
    def _stage_rel():
        rtab_cp.append(pltpu.async_copy(rel_hbm, rtab, isem))

    ci.wait()
    cj.wait()
    ck.wait()

    def fire_ht(i):
        start, n = CHUNKS[i]
        buf = i % NBUF
        s = pl.ds(start, n)
        d = pl.ds(0, n)
        return (
            pltpu.async_copy(ent_hbm.at[hidx.at[s]], rows.at[buf, 0].at[d],
                             hsems[buf]),
            pltpu.async_copy(ent_hbm.at[tidx.at[s]], rows.at[buf, 1].at[d],
                             tsems[buf]),
        )

    def fire_radd(i):
        start, n = CHUNKS[i]
        buf = i % NBUF
        return pltpu.async_copy(rtab.at[ridx.at[pl.ds(start, n)]],
                                rows.at[buf, 0].at[pl.ds(0, n)],
                                rsems[buf], add=True)

    ht = {0: fire_ht(0), 1: fire_ht(1)}

    @pl.when(lax.axis_index("s") == 0)
    def _wait_rel():
        rtab_cp[0].wait()

    plsc.subcore_barrier()
    ht[0][0].wait()
    radd = {0: fire_radd(0)}

    outcps = []
    for i in range(len(CHUNKS)):
        start, n = CHUNKS[i]
        radd[i].wait()
        ht[i][1].wait()
        if i + 2 < len(CHUNKS):
            ht[i + 2] = fire_ht(i + 2)
        if i + 1 < len(CHUNKS):
            ht[i + 1][0].wait()
            radd[i + 1] = fire_radd(i + 1)
        buf = i % NBUF
        hrow = rows.at[buf, 0]
        trow = rows.at[buf, 1]

        @plsc.parallel_loop(0, n // GRP, unroll=2)
        def group(g):
            e0 = g * GRP
            res = jnp.zeros((16,), jnp.float32)
            for u in range(GRP):
                e = e0 + u
                acc_a = None
                acc_b = None
                for k in range(DIM // 16):
                    sl = pl.ds(k * 16, 16)
                    term = jnp.abs(hrow[e, sl] - trow[e, sl])
                    if k % 2 == 0:
                        acc_a = term if acc_a is None else acc_a + term
                    else:
                        acc_b = term if acc_b is None else acc_b + term
                tot = jnp.sum(acc_a + acc_b)
                res = jnp.where(lane == u, tot, res)
            plsc.store_scatter(outb, [start + e0 + lane], res,
                               mask=lane < GRP)

        outcps.append(pltpu.async_copy(
            outb.at[pl.ds(start, n)],
            out_hbm.at[pl.ds(base + start, n)], isem))

    for d in outcps:
        d.wait()


@jax.jit
def kernel(heads, rels, tails, ent_embs, rel_embs):
    mesh = plsc.VectorSubcoreMesh(core_axis_name="c", subcore_axis_name="s")
    f = functools.partial(
        pl.kernel,
        mesh=mesh,
        compiler_params=pltpu.CompilerParams(needs_layout_passes=False),
        out_type=jax.ShapeDtypeStruct((BATCH,), jnp.float32),
        scratch_types=[
            pltpu.VMEM((B_PER_W,), jnp.int32),
            pltpu.VMEM((B_PER_W,), jnp.int32),
            pltpu.VMEM((B_PER_W,), jnp.int32),
            pltpu.VMEM((NBUF, 2, CHUNK, DIM), jnp.float32),
            pltpu.VMEM_SHARED((1000, DIM), jnp.float32),
            pltpu.VMEM((B_PER_W,), jnp.float32),
            pltpu.SemaphoreType.DMA,
            pltpu.SemaphoreType.DMA,
            pltpu.SemaphoreType.DMA,
            pltpu.SemaphoreType.DMA,
            pltpu.SemaphoreType.DMA,
            pltpu.SemaphoreType.DMA,
            pltpu.SemaphoreType.DMA,
            pltpu.SemaphoreType.DMA,
            pltpu.SemaphoreType.DMA,
            pltpu.SemaphoreType.DMA,
        ],
    )(_body)
    return f(heads, rels, tails, ent_embs, rel_embs)
